# baseline (speedup 1.0000x reference)
import functools

import jax
import jax.numpy as jnp
from jax import lax
from jax.experimental import pallas as pl
from jax.experimental.pallas import tpu as pltpu
from jax.experimental.pallas import tpu_sc as plsc

NC = 2
NS = 16
L = 16
NW = NC * NS
BINS = 64
ROW = 2 * BINS
CH = 32768
NBUF = 3
UNROLL = 8


@functools.lru_cache(maxsize=None)
def _make_sc_hist(shape):
    b, c, h, w = shape
    assert b == NW and (c * h * w) % CH == 0 and w % L == 0
    rb = CH // w
    nch = (c * h) // rb
    assert h % rb == 0 and nch % 2 == 0

    mesh = plsc.VectorSubcoreMesh(core_axis_name="c", subcore_axis_name="s")

    @functools.partial(
        pl.kernel,
        out_type=jax.ShapeDtypeStruct((NW * ROW,), jnp.float32),
        mesh=mesh,
        compiler_params=pltpu.CompilerParams(needs_layout_passes=False),
        scratch_types=[
            [pltpu.VMEM((CH // 512, 512), jnp.float32)] * NBUF,
            pltpu.VMEM((UNROLL * L * ROW,), jnp.float32),
            pltpu.VMEM((ROW,), jnp.float32),
            [pltpu.SemaphoreType.DMA] * NBUF,
        ],
    )
    def sc_hist(img1, img2, out, bufs, hist, orow, sems):
        wid = lax.axis_index("s") * NC + lax.axis_index("c")
        bpp = h // rb

        def csrc(img, t):
            return img.at[wid, t // bpp, pl.ds((t % bpp) * rb, rb), :]

        zero = jnp.zeros((L,), jnp.float32)

        def zb(i, _):
            hist[pl.ds(i * L, L)] = zero
            return 0

        lax.fori_loop(0, UNROLL * ROW, zb, 0)

        lane_iota = lax.iota(jnp.int32, L)
        ones = jnp.ones((L,), jnp.float32)

        vpr = w // L

        def proc(buf, lbcs):
            @plsc.parallel_loop(0, CH // L, step=UNROLL)
            def body(v):
                for u in range(UNROLL):
                    vv = v + u
                    x = buf[vv // vpr, pl.ds((vv % vpr) * L, L)]
                    idx = jnp.minimum(x * 64.0, float(BINS - 1)).astype(
                        jnp.int32
                    )
                    plsc.addupdate_scatter(hist, [(idx << 4) + lbcs[u]], ones)

        assert nch % NBUF == 0
        for img, boff in ((img1, 0), (img2, BINS)):
            lbcs = [lane_iota + (boff * L + u * L * ROW) for u in range(UNROLL)]
            for j in range(NBUF):
                pltpu.async_copy(csrc(img, j), bufs[j], sems[j])

            def grp(k, _):
                t0 = NBUF * k
                for j in range(NBUF):
                    pltpu.make_async_copy(csrc(img, 0), bufs[j], sems[j]).wait()
                    proc(bufs[j], lbcs)

                    @pl.when(t0 + j + NBUF < nch)
                    def _():
                        pltpu.async_copy(
                            csrc(img, t0 + j + NBUF), bufs[j], sems[j]
                        )

                return 0

            lax.fori_loop(0, nch // NBUF, grp, 0)

        def red(j, _):
            def redbin(t, acc):
                b = j * L + t

                def redbank(u, v):
                    return v + hist[pl.ds(u * (L * ROW) + b * L, L)]

                v = lax.fori_loop(0, UNROLL, redbank, jnp.zeros((L,), jnp.float32))
                return jnp.where(lane_iota == t, jnp.sum(v), acc)

            orow[pl.ds(j * L, L)] = lax.fori_loop(
                0, L, redbin, jnp.zeros((L,), jnp.float32)
            )
            return 0

        lax.fori_loop(0, ROW // L, red, 0)
        pltpu.sync_copy(orow, out.at[pl.ds(wid * ROW, ROW)])

    return sc_hist


def _l1n(v, eps=1e-12):
    n = jnp.sum(jnp.abs(v), axis=-1, keepdims=True)
    return v / jnp.maximum(n, eps)


def _kl(p, q):
    p = _l1n(p)
    q = _l1n(q)
    return jnp.sum(p * jnp.log(p / (q + 1e-08) + 1e-08), axis=-1)


def kernel(imgl, img2, bins):
    del bins
    b, c, h, w = imgl.shape
    rows = _make_sc_hist(imgl.shape)(imgl, img2).reshape(NW, ROW)
    s = jnp.sum(rows, axis=0)
    hist1 = s[:BINS] / (h * w)
    hist2 = s[BINS:] / (h * w)
    loss = _kl(hist1, hist2) + _kl(hist2, hist1)
    return jnp.mean(loss)


# speedup vs baseline: 1.0091x; 1.0091x over previous
import functools

import jax
import jax.numpy as jnp
from jax import lax
from jax.experimental import pallas as pl
from jax.experimental.pallas import tpu as pltpu
from jax.experimental.pallas import tpu_sc as plsc

NC = 2
NS = 16
L = 16
NW = NC * NS
BINS = 64
ROW = 2 * BINS
CH = 32768
NBUF = 2
UNROLL = 16


@functools.lru_cache(maxsize=None)
def _make_sc_hist(shape):
    b, c, h, w = shape
    assert b == NW and (c * h * w) % CH == 0 and w % L == 0
    rb = CH // w
    nch = (c * h) // rb
    assert h % rb == 0 and nch % 2 == 0

    mesh = plsc.VectorSubcoreMesh(core_axis_name="c", subcore_axis_name="s")

    @functools.partial(
        pl.kernel,
        out_type=jax.ShapeDtypeStruct((NW * ROW,), jnp.float32),
        mesh=mesh,
        compiler_params=pltpu.CompilerParams(needs_layout_passes=False),
        scratch_types=[
            [pltpu.VMEM((CH // 512, 512), jnp.float32)] * NBUF,
            pltpu.VMEM((UNROLL * L * ROW,), jnp.float32),
            pltpu.VMEM((ROW,), jnp.float32),
            [pltpu.SemaphoreType.DMA] * NBUF,
        ],
    )
    def sc_hist(img1, img2, out, bufs, hist, orow, sems):
        wid = lax.axis_index("s") * NC + lax.axis_index("c")
        bpp = h // rb

        def csrc(img, t):
            return img.at[wid, t // bpp, pl.ds((t % bpp) * rb, rb), :]

        zero = jnp.zeros((L,), jnp.float32)

        def zb(i, _):
            hist[pl.ds(i * L, L)] = zero
            return 0

        lax.fori_loop(0, UNROLL * ROW, zb, 0)

        lane_iota = lax.iota(jnp.int32, L)
        ones = jnp.ones((L,), jnp.float32)

        vpr = w // L

        def proc(buf, lbcs):
            @plsc.parallel_loop(0, CH // L, step=UNROLL)
            def body(v):
                for u in range(UNROLL):
                    vv = v + u
                    x = buf[vv // vpr, pl.ds((vv % vpr) * L, L)]
                    idx = jnp.minimum(x * 64.0, float(BINS - 1)).astype(
                        jnp.int32
                    )
                    plsc.addupdate_scatter(hist, [(idx << 4) + lbcs[u]], ones)

        assert nch % NBUF == 0
        for img, boff in ((img1, 0), (img2, BINS)):
            lbcs = [lane_iota + (boff * L + u * L * ROW) for u in range(UNROLL)]
            for j in range(NBUF):
                pltpu.async_copy(csrc(img, j), bufs[j], sems[j])

            def grp(k, _):
                t0 = NBUF * k
                for j in range(NBUF):
                    pltpu.make_async_copy(csrc(img, 0), bufs[j], sems[j]).wait()
                    proc(bufs[j], lbcs)

                    @pl.when(t0 + j + NBUF < nch)
                    def _():
                        pltpu.async_copy(
                            csrc(img, t0 + j + NBUF), bufs[j], sems[j]
                        )

                return 0

            lax.fori_loop(0, nch // NBUF, grp, 0)

        def red(j, _):
            def redbin(t, acc):
                b = j * L + t

                def redbank(u, v):
                    return v + hist[pl.ds(u * (L * ROW) + b * L, L)]

                v = lax.fori_loop(0, UNROLL, redbank, jnp.zeros((L,), jnp.float32))
                return jnp.where(lane_iota == t, jnp.sum(v), acc)

            orow[pl.ds(j * L, L)] = lax.fori_loop(
                0, L, redbin, jnp.zeros((L,), jnp.float32)
            )
            return 0

        lax.fori_loop(0, ROW // L, red, 0)
        pltpu.sync_copy(orow, out.at[pl.ds(wid * ROW, ROW)])

    return sc_hist


def _l1n(v, eps=1e-12):
    n = jnp.sum(jnp.abs(v), axis=-1, keepdims=True)
    return v / jnp.maximum(n, eps)


def _kl(p, q):
    p = _l1n(p)
    q = _l1n(q)
    return jnp.sum(p * jnp.log(p / (q + 1e-08) + 1e-08), axis=-1)


def kernel(imgl, img2, bins):
    del bins
    b, c, h, w = imgl.shape
    rows = _make_sc_hist(imgl.shape)(imgl, img2).reshape(NW, ROW)
    s = jnp.sum(rows, axis=0)
    hist1 = s[:BINS] / (h * w)
    hist2 = s[BINS:] / (h * w)
    loss = _kl(hist1, hist2) + _kl(hist2, hist1)
    return jnp.mean(loss)


# speedup vs baseline: 1.0188x; 1.0097x over previous
import functools

import jax
import jax.numpy as jnp
from jax import lax
from jax.experimental import pallas as pl
from jax.experimental.pallas import tpu as pltpu
from jax.experimental.pallas import tpu_sc as plsc

NC = 2
NS = 16
L = 16
NW = NC * NS
BINS = 64
ROW = 2 * BINS
CH = 32768
NBUF = 2
UNROLL = 8


@functools.lru_cache(maxsize=None)
def _make_sc_hist(shape):
    b, c, h, w = shape
    assert b == NW and (c * h * w) % CH == 0 and w % L == 0
    rb = CH // w
    nch = (c * h) // rb
    assert h % rb == 0 and nch % 2 == 0

    mesh = plsc.VectorSubcoreMesh(core_axis_name="c", subcore_axis_name="s")

    @functools.partial(
        pl.kernel,
        out_type=jax.ShapeDtypeStruct((NW * ROW,), jnp.float32),
        mesh=mesh,
        compiler_params=pltpu.CompilerParams(needs_layout_passes=False),
        scratch_types=[
            [pltpu.VMEM((CH // 512, 512), jnp.float32)] * NBUF,
            pltpu.VMEM((UNROLL * L * ROW,), jnp.float32),
            pltpu.VMEM((ROW,), jnp.float32),
            [pltpu.SemaphoreType.DMA] * NBUF,
        ],
    )
    def sc_hist(img1, img2, out, bufs, hist, orow, sems):
        wid = lax.axis_index("s") * NC + lax.axis_index("c")
        bpp = h // rb

        def csrc(img, t):
            return img.at[wid, t // bpp, pl.ds((t % bpp) * rb, rb), :]

        zero = jnp.zeros((L,), jnp.float32)

        def zb(i, _):
            hist[pl.ds(i * L, L)] = zero
            return 0

        lax.fori_loop(0, UNROLL * ROW, zb, 0)

        lane_iota = lax.iota(jnp.int32, L)
        ones = jnp.ones((L,), jnp.float32)

        vpr = w // L

        def proc(buf, lbcs):
            @plsc.parallel_loop(0, CH // L, step=UNROLL)
            def body(v):
                for u in range(UNROLL):
                    vv = v + u
                    x = buf[vv // vpr, pl.ds((vv % vpr) * L, L)]
                    idx = jnp.minimum(x * 64.0, float(BINS - 1)).astype(
                        jnp.int32
                    )
                    plsc.addupdate_scatter(hist, [(idx << 4) + lbcs[u]], ones)

        assert nch % NBUF == 0
        for img, boff in ((img1, 0), (img2, BINS)):
            lbcs = [lane_iota + (boff * L + u * L * ROW) for u in range(UNROLL)]
            for j in range(NBUF):
                pltpu.async_copy(csrc(img, j), bufs[j], sems[j])

            def grp(k, _):
                t0 = NBUF * k
                for j in range(NBUF):
                    pltpu.make_async_copy(csrc(img, 0), bufs[j], sems[j]).wait()
                    proc(bufs[j], lbcs)

                    @pl.when(t0 + j + NBUF < nch)
                    def _():
                        pltpu.async_copy(
                            csrc(img, t0 + j + NBUF), bufs[j], sems[j]
                        )

                return 0

            lax.fori_loop(0, nch // NBUF, grp, 0)

        def red(j, _):
            def redbin(t, acc):
                b = j * L + t

                def redbank(u, v):
                    return v + hist[pl.ds(u * (L * ROW) + b * L, L)]

                v = lax.fori_loop(0, UNROLL, redbank, jnp.zeros((L,), jnp.float32))
                return jnp.where(lane_iota == t, jnp.sum(v), acc)

            orow[pl.ds(j * L, L)] = lax.fori_loop(
                0, L, redbin, jnp.zeros((L,), jnp.float32)
            )
            return 0

        lax.fori_loop(0, ROW // L, red, 0)
        pltpu.sync_copy(orow, out.at[pl.ds(wid * ROW, ROW)])

    return sc_hist


def _l1n(v, eps=1e-12):
    n = jnp.sum(jnp.abs(v), axis=-1, keepdims=True)
    return v / jnp.maximum(n, eps)


def _kl(p, q):
    p = _l1n(p)
    q = _l1n(q)
    return jnp.sum(p * jnp.log(p / (q + 1e-08) + 1e-08), axis=-1)


def kernel(imgl, img2, bins):
    del bins
    b, c, h, w = imgl.shape
    rows = _make_sc_hist(imgl.shape)(imgl, img2).reshape(NW, ROW)
    s = jnp.sum(rows, axis=0)
    hist1 = s[:BINS] / (h * w)
    hist2 = s[BINS:] / (h * w)
    loss = _kl(hist1, hist2) + _kl(hist2, hist1)
    return jnp.mean(loss)
